# baseline (device time: 11820 ns/iter reference)
import jax
import jax.numpy as jnp
from jax import lax
from jax.experimental import pallas as pl
from jax.experimental.pallas import tpu as pltpu

N_DEV = 4
N_WAVES = 2


def kernel(x):
    m_per, n = x.shape
    m_global = N_DEV * m_per
    m_half = m_per // N_WAVES

    def body(x_ref, out_ref, send_buf, comm_ref, send_sems, recv_sems):
        my_pos = lax.axis_index("i")

        barrier_sem = pltpu.get_barrier_semaphore()
        for off in range(1, N_DEV):
            pl.semaphore_signal(
                barrier_sem, inc=1,
                device_id=((my_pos + off) % N_DEV,),
                device_id_type=pl.DeviceIdType.MESH,
            )

        def start_wave(w):
            sends = []
            for off in range(1, N_DEV):
                rdma = pltpu.make_async_remote_copy(
                    src_ref=send_buf.at[w],
                    dst_ref=comm_ref.at[w, off - 1],
                    send_sem=send_sems.at[w, off - 1],
                    recv_sem=recv_sems.at[w, off - 1],
                    device_id=((my_pos + off) % N_DEV,),
                    device_id_type=pl.DeviceIdType.MESH,
                )
                rdma.start()
                sends.append(rdma)
            return sends

        p0 = jnp.sum(x_ref[0:m_half, :], axis=0, keepdims=True)
        send_buf[0] = p0
        pl.semaphore_wait(barrier_sem, N_DEV - 1)
        sends = start_wave(0)

        p1 = jnp.sum(x_ref[m_half:, :], axis=0, keepdims=True)
        send_buf[1] = p1
        sends += start_wave(1)

        total = p0 + p1
        for w in range(N_WAVES):
            for slot in range(N_DEV - 1):
                recv = pltpu.make_async_remote_copy(
                    src_ref=send_buf.at[w],
                    dst_ref=comm_ref.at[w, slot],
                    send_sem=send_sems.at[w, slot],
                    recv_sem=recv_sems.at[w, slot],
                    device_id=(my_pos,),
                    device_id_type=pl.DeviceIdType.MESH,
                )
                recv.wait_recv()
                total = total + comm_ref[w, slot]

        out_ref[:, :] = total * (1.0 / m_global)

        for rdma in sends:
            rdma.wait_send()

    return pl.pallas_call(
        body,
        out_shape=jax.ShapeDtypeStruct((1, n), jnp.float32),
        in_specs=[pl.BlockSpec(memory_space=pltpu.VMEM)],
        out_specs=pl.BlockSpec(memory_space=pltpu.VMEM),
        scratch_shapes=[
            pltpu.VMEM((N_WAVES, 1, n), jnp.float32),
            pltpu.VMEM((N_WAVES, N_DEV - 1, 1, n), jnp.float32),
            pltpu.SemaphoreType.DMA((N_WAVES, N_DEV - 1)),
            pltpu.SemaphoreType.DMA((N_WAVES, N_DEV - 1)),
        ],
        compiler_params=pltpu.CompilerParams(collective_id=0),
    )(x)
